# baseline (speedup 1.0000x reference)
import functools

import jax
import jax.numpy as jnp
from jax import lax
from jax.experimental import pallas as pl
from jax.experimental.pallas import tpu as pltpu
from jax.experimental.pallas import tpu_sc as plsc

_B = 16384

_NC = 2
_NS = 16
_NW = _NC * _NS
_BPW = _B // _NW
_CHUNK = 128
_NCHUNK = _BPW // _CHUNK
_L = 16


def _sc_body(x_hbm, w_hbm, out_hbm, xv, idxv, outv, sem):
    wid = lax.axis_index("s") * _NC + lax.axis_index("c")
    base = wid * _BPW

    pltpu.sync_copy(x_hbm.at[pl.ds(base * 3, _BPW * 3)], xv)

    lane_iota = lax.iota(jnp.int32, _L)
    for g in range(_BPW // _L):
        t = (lane_iota + g * _L) * 3
        i0 = plsc.load_gather(xv, [t])
        i1 = plsc.load_gather(xv, [t + 1])
        i2 = plsc.load_gather(xv, [t + 2])
        j, k = divmod(g, _CHUNK // _L)
        idxv[j, pl.ds(k * _L, _L)] = i0 * 10000 + i1 * 100 + i2

    copies = [
        pltpu.async_copy(
            w_hbm.at[idxv.at[j]],
            outv.at[pl.ds(j * _CHUNK, _CHUNK)],
            sem,
        )
        for j in range(_NCHUNK)
    ]
    for c in copies:
        c.wait()

    pltpu.sync_copy(outv, out_hbm.at[pl.ds(base, _BPW)])


@functools.partial(jax.jit)
def _sc_gather(x_flat, w_flat):
    mesh = plsc.VectorSubcoreMesh(core_axis_name="c", subcore_axis_name="s")
    return pl.kernel(
        _sc_body,
        mesh=mesh,
        compiler_params=pltpu.CompilerParams(needs_layout_passes=False),
        out_type=jax.ShapeDtypeStruct((_B,), jnp.float32),
        scratch_types=[
            pltpu.VMEM((3 * _BPW,), jnp.int32),
            pltpu.VMEM((_NCHUNK, _CHUNK), jnp.int32),
            pltpu.VMEM((_BPW,), jnp.float32),
            pltpu.SemaphoreType.DMA,
        ],
    )(x_flat, w_flat)


def kernel(x, W):
    x_flat = x.astype(jnp.int32).reshape(-1)
    w_flat = W.reshape(-1)
    return _sc_gather(x_flat, w_flat).reshape(_B, 1)


# speedup vs baseline: 1.3197x; 1.3197x over previous
import functools

import jax
import jax.numpy as jnp
from jax import lax
from jax.experimental import pallas as pl
from jax.experimental.pallas import tpu as pltpu
from jax.experimental.pallas import tpu_sc as plsc

_B = 16384

_NC = 2
_NS = 16
_NW = _NC * _NS
_BPW = _B // _NW
_CHUNK = 128
_NCHUNK = _BPW // _CHUNK
_L = 16


def _sc_body(x_hbm, w_hbm, out_hbm, xv, idxv, outv, sem):
    wid = lax.axis_index("s") * _NC + lax.axis_index("c")
    base = wid * _BPW

    for d in range(3):
        pltpu.sync_copy(
            x_hbm.at[pl.ds(d * _B + base, _BPW)], xv.at[pl.ds(d * _BPW, _BPW)]
        )

    for g in range(_BPW // _L):
        i0 = xv[pl.ds(g * _L, _L)]
        i1 = xv[pl.ds(_BPW + g * _L, _L)]
        i2 = xv[pl.ds(2 * _BPW + g * _L, _L)]
        j, k = divmod(g, _CHUNK // _L)
        idxv[j, pl.ds(k * _L, _L)] = i0 * 10000 + i1 * 100 + i2

    copies = [
        pltpu.async_copy(
            w_hbm.at[idxv.at[j]],
            outv.at[pl.ds(j * _CHUNK, _CHUNK)],
            sem,
        )
        for j in range(_NCHUNK)
    ]
    for c in copies:
        c.wait()

    pltpu.sync_copy(outv, out_hbm.at[pl.ds(base, _BPW)])


@functools.partial(jax.jit)
def _sc_gather(x_flat, w_flat):
    mesh = plsc.VectorSubcoreMesh(core_axis_name="c", subcore_axis_name="s")
    return pl.kernel(
        _sc_body,
        mesh=mesh,
        compiler_params=pltpu.CompilerParams(needs_layout_passes=False),
        out_type=jax.ShapeDtypeStruct((_B,), jnp.float32),
        scratch_types=[
            pltpu.VMEM((3 * _BPW,), jnp.int32),
            pltpu.VMEM((_NCHUNK, _CHUNK), jnp.int32),
            pltpu.VMEM((_BPW,), jnp.float32),
            pltpu.SemaphoreType.DMA,
        ],
    )(x_flat, w_flat)


def kernel(x, W):
    x_flat = x.astype(jnp.int32).T.reshape(-1)
    w_flat = W.reshape(-1)
    return _sc_gather(x_flat, w_flat).reshape(_B, 1)
